# baseline (speedup 1.0000x reference)
from functools import partial

import jax
import jax.numpy as jnp
from jax import lax
from jax.experimental import pallas as pl
from jax.experimental.pallas import tpu as pltpu

_BB = 512


def _lstm_kernel(x_ref, wih_ref, whh_ref, bih_ref, bhh_ref, wout_ref,
                 bout_ref, out_ref, wcat_s, wout_s, *, T: int):
    BB = x_ref.shape[0]
    H4 = wih_ref.shape[1]
    H = H4 // 4

    @pl.when(pl.program_id(0) == 0)
    def _prep():
        lane = lax.broadcasted_iota(jnp.int32, (1, H4), 1)
        cs = jnp.where((lane >= 2 * H) & (lane < 3 * H), 1.0, 0.5)
        bias = ((bih_ref[...] + bhh_ref[...]) * cs).astype(jnp.bfloat16)
        wcat_s[...] = jnp.concatenate(
            [(wih_ref[...] * cs).astype(jnp.bfloat16),
             (whh_ref[...] * cs).astype(jnp.bfloat16),
             jnp.broadcast_to(bias, (8, H4))], axis=0)
        wout_s[...] = wout_ref[...].astype(jnp.bfloat16)

    w_cat = wcat_s[...]

    xb = x_ref[...].astype(jnp.bfloat16)
    xs = [xb[:, t, :] for t in range(T)]

    okc = jnp.full((BB, 8), 0.125, jnp.bfloat16)

    h = jnp.zeros((BB, H), jnp.bfloat16)
    c = jnp.zeros((BB, H), jnp.float32)
    for t in range(T):
        xh = jnp.concatenate([xs[t], h, okc], axis=1)
        gates = jnp.dot(xh, w_cat, preferred_element_type=jnp.float32)
        a = jnp.tanh(gates)
        i_g = a[:, 0 * H:1 * H] * 0.5 + 0.5
        f_g = a[:, 1 * H:2 * H] * 0.5 + 0.5
        g_g = a[:, 2 * H:3 * H]
        o_g = a[:, 3 * H:4 * H] * 0.5 + 0.5
        c = f_g * c + i_g * g_g
        h = (o_g * jnp.tanh(c)).astype(jnp.bfloat16)

    out_ref[...] = (jnp.dot(h, wout_s[...],
                            preferred_element_type=jnp.float32)
                    + bout_ref[...]).astype(out_ref.dtype)


def kernel(x, w_ih, w_hh, b_ih, b_hh, w_out, b_out):
    B, T, D = x.shape
    H = w_hh.shape[0]
    A = w_out.shape[1]
    H4 = 4 * H

    BB = min(_BB, B)
    nb = -(-B // BB)
    Bp = nb * BB
    if Bp != B:
        x = jnp.pad(x, ((0, Bp - B), (0, 0), (0, 0)))

    out_p = pl.pallas_call(
        partial(_lstm_kernel, T=T),
        out_shape=jax.ShapeDtypeStruct((Bp, A), jnp.float32),
        grid=(nb,),
        in_specs=[
            pl.BlockSpec((BB, T, D), lambda i: (i, 0, 0)),
            pl.BlockSpec((D, H4), lambda i: (0, 0)),
            pl.BlockSpec((H, H4), lambda i: (0, 0)),
            pl.BlockSpec((1, H4), lambda i: (0, 0)),
            pl.BlockSpec((1, H4), lambda i: (0, 0)),
            pl.BlockSpec((H, A), lambda i: (0, 0)),
            pl.BlockSpec((1, A), lambda i: (0, 0)),
        ],
        out_specs=pl.BlockSpec((BB, A), lambda i: (i, 0)),
        scratch_shapes=[
            pltpu.VMEM((D + H + 8, H4), jnp.bfloat16),
            pltpu.VMEM((H, A), jnp.bfloat16),
        ],
        compiler_params=pltpu.CompilerParams(
            dimension_semantics=("arbitrary",)),
    )(x, w_ih, w_hh, b_ih[None, :], b_hh[None, :], w_out, b_out[None, :])
    return out_p[:B]


# speedup vs baseline: 1.2186x; 1.2186x over previous
from functools import partial

import jax
import jax.numpy as jnp
from jax import lax
from jax.experimental import pallas as pl
from jax.experimental.pallas import tpu as pltpu

_BB = 1024


def _lstm_kernel(x_ref, wih_ref, whh_ref, bih_ref, bhh_ref, wout_ref,
                 bout_ref, out_ref, *, T: int):
    BB = x_ref.shape[0]
    H4 = wih_ref.shape[1]
    H = H4 // 4

    lane = lax.broadcasted_iota(jnp.int32, (1, H4), 1)
    cs = jnp.where((lane >= 2 * H) & (lane < 3 * H), 1.0, 0.5)
    bias = ((bih_ref[...] + bhh_ref[...]) * cs).astype(jnp.bfloat16)
    w_cat = jnp.concatenate(
        [(wih_ref[...] * cs).astype(jnp.bfloat16),
         (whh_ref[...] * cs).astype(jnp.bfloat16),
         jnp.broadcast_to(bias, (8, H4)).astype(jnp.bfloat16)],
        axis=0)

    xb = x_ref[...].astype(jnp.bfloat16)
    xs = [xb[:, t, :] for t in range(T)]

    okc = jnp.full((BB, 8), 0.125, jnp.bfloat16)

    h = jnp.zeros((BB, H), jnp.float32)
    c = jnp.zeros((BB, H), jnp.float32)
    for t in range(T):
        xh = jnp.concatenate([xs[t], h.astype(jnp.bfloat16), okc], axis=1)
        gates = jnp.dot(xh, w_cat, preferred_element_type=jnp.float32)
        a = jnp.tanh(gates)
        i_g = a[:, 0 * H:1 * H] * 0.5 + 0.5
        f_g = a[:, 1 * H:2 * H] * 0.5 + 0.5
        g_g = a[:, 2 * H:3 * H]
        o_g = a[:, 3 * H:4 * H] * 0.5 + 0.5
        c = f_g * c + i_g * g_g
        h = o_g * jnp.tanh(c)

    out_ref[...] = (jnp.dot(h, wout_ref[...],
                            preferred_element_type=jnp.float32)
                    + bout_ref[...]).astype(out_ref.dtype)


def kernel(x, w_ih, w_hh, b_ih, b_hh, w_out, b_out):
    B, T, D = x.shape
    H = w_hh.shape[0]
    A = w_out.shape[1]
    H4 = 4 * H

    BB = min(_BB, B)
    nb = -(-B // BB)
    Bp = nb * BB
    if Bp != B:
        x = jnp.pad(x, ((0, Bp - B), (0, 0), (0, 0)))

    out_p = pl.pallas_call(
        partial(_lstm_kernel, T=T),
        out_shape=jax.ShapeDtypeStruct((Bp, A), jnp.float32),
        grid=(nb,),
        in_specs=[
            pl.BlockSpec((BB, T, D), lambda i: (i, 0, 0)),
            pl.BlockSpec((D, H4), lambda i: (0, 0)),
            pl.BlockSpec((H, H4), lambda i: (0, 0)),
            pl.BlockSpec((1, H4), lambda i: (0, 0)),
            pl.BlockSpec((1, H4), lambda i: (0, 0)),
            pl.BlockSpec((H, A), lambda i: (0, 0)),
            pl.BlockSpec((1, A), lambda i: (0, 0)),
        ],
        out_specs=pl.BlockSpec((BB, A), lambda i: (i, 0)),
        compiler_params=pltpu.CompilerParams(
            dimension_semantics=("arbitrary",)),
    )(x, w_ih, w_hh, b_ih[None, :], b_hh[None, :], w_out, b_out[None, :])
    return out_p[:B]
